# Initial kernel scaffold; baseline (speedup 1.0000x reference)
import jax
import jax.numpy as jnp
from jax.experimental import pallas as pl


def kernel(isbn_idx, author_idx, year_of_publication, book_table, author_table, year_table, boundaries, year_mean, year_std):
    raise NotImplementedError("write your pallas kernel here")


# speedup vs baseline: 1.4363x; 1.4363x over previous
import functools

import jax
import jax.numpy as jnp
from jax import lax
from jax.experimental import pallas as pl
from jax.experimental.pallas import tpu as pltpu
from jax.experimental.pallas import tpu_sc as plsc

_L = 16
_CH = 128


@functools.lru_cache(maxsize=None)
def _build(B, D, NB):
    info = plsc.get_sparse_core_info()
    NC, NS = info.num_cores, info.num_subcores
    NW = NC * NS
    assert B % NW == 0
    bpw = B // NW
    assert bpw % _CH == 0
    ng = bpw // _CH
    W = 3 * D + 1
    mesh = plsc.VectorSubcoreMesh(core_axis_name="c", subcore_axis_name="s")

    @functools.partial(
        pl.kernel,
        mesh=mesh,
        compiler_params=pltpu.CompilerParams(
            needs_layout_passes=False, use_tc_tiling_on_sc=False),
        out_type=jax.ShapeDtypeStruct((B * W,), jnp.float32),
        scratch_types=[
            pltpu.VMEM((bpw,), jnp.int32),
            pltpu.VMEM((bpw,), jnp.int32),
            pltpu.VMEM((bpw,), jnp.float32),
            pltpu.VMEM((bpw,), jnp.int32),
            pltpu.VMEM((bpw,), jnp.float32),
            pltpu.VMEM((NB + 4,), jnp.float32),
            pltpu.VMEM((2 * _L,), jnp.float32),
            pltpu.VMEM((bpw, D), jnp.float32),
            pltpu.VMEM((bpw, D), jnp.float32),
            pltpu.VMEM((bpw, D), jnp.float32),
            pltpu.VMEM((_CH * W,), jnp.float32),
            pltpu.SemaphoreType.DMA,
            pltpu.SemaphoreType.DMA,
            pltpu.SemaphoreType.DMA,
        ],
    )
    def k(isbn_hbm, auth_hbm, year_hbm, btab_hbm, atab_hbm, ytab_hbm,
          bpad_hbm, consts_hbm, out_hbm,
          bidx_v, aidx_v, year_v, ybkt_v, ny_v, bpad_v, consts_v,
          brows_v, arows_v, yrows_v, blk_v,
          sem0, sem1, sem2):
        wid = lax.axis_index("s") * NC + lax.axis_index("c")
        base = wid * bpw

        c0 = pltpu.async_copy(isbn_hbm.at[pl.ds(base, bpw)], bidx_v, sem0)
        c1 = pltpu.async_copy(auth_hbm.at[pl.ds(base, bpw)], aidx_v, sem1)
        c2 = pltpu.async_copy(year_hbm.at[pl.ds(base, bpw)], year_v, sem2)
        pltpu.sync_copy(bpad_hbm, bpad_v)
        pltpu.sync_copy(consts_hbm, consts_v)
        c0.wait()
        c1.wait()
        c2.wait()

        cps = []
        for g in range(ng):
            cps.append(pltpu.async_copy(
                btab_hbm.at[bidx_v.at[pl.ds(g * _CH, _CH)]],
                brows_v.at[pl.ds(g * _CH, _CH)], sem0))
            cps.append(pltpu.async_copy(
                atab_hbm.at[aidx_v.at[pl.ds(g * _CH, _CH)]],
                arows_v.at[pl.ds(g * _CH, _CH)], sem1))

        mean = consts_v[pl.ds(0, _L)]
        std = consts_v[pl.ds(_L, _L)]
        scale = jnp.float32(NB - 1)
        for c in range(bpw // _L):
            y = year_v[pl.ds(c * _L, _L)]
            j = jnp.clip((y * scale).astype(jnp.int32) + 1, 0, NB)
            for _ in range(2):
                hi = plsc.load_gather(bpad_v, [j + 1])
                lo = plsc.load_gather(bpad_v, [j])
                j = j + jnp.where(hi <= y, 1, 0) - jnp.where(lo > y, 1, 0)
            ybkt_v[pl.ds(c * _L, _L)] = j
            ny_v[pl.ds(c * _L, _L)] = (y - mean) / std

        for g in range(ng):
            cps.append(pltpu.async_copy(
                ytab_hbm.at[ybkt_v.at[pl.ds(g * _CH, _CH)]],
                yrows_v.at[pl.ds(g * _CH, _CH)], sem2))
        for cp in cps:
            cp.wait()

        lane = jnp.arange(_L, dtype=jnp.int32)
        for g in range(ng):
            def row_body(r, _, g=g):
                src = g * _CH + r
                dst = r * W
                for t, buf in ((0, brows_v), (1, arows_v), (2, yrows_v)):
                    for c4 in range(D // _L):
                        v = buf[src, pl.ds(c4 * _L, _L)]
                        plsc.store_scatter(
                            blk_v, [dst + (t * D + c4 * _L) + lane], v)
                return 0

            lax.fori_loop(0, _CH, row_body, 0)
            for r0 in range(0, _CH, _L):
                v = ny_v[pl.ds(g * _CH + r0, _L)]
                plsc.store_scatter(blk_v, [(r0 + lane) * W + 3 * D], v)
            pltpu.sync_copy(
                blk_v, out_hbm.at[pl.ds((base + g * _CH) * W, _CH * W)])

    return k


def kernel(isbn_idx, author_idx, year_of_publication, book_table,
           author_table, year_table, boundaries, year_mean, year_std):
    B = isbn_idx.shape[0]
    D = book_table.shape[1]
    NB = boundaries.shape[0]
    k = _build(B, D, NB)
    neg = jnp.full((1,), -jnp.inf, dtype=jnp.float32)
    pos = jnp.full((3,), jnp.inf, dtype=jnp.float32)
    bpad = jnp.concatenate([neg, boundaries.astype(jnp.float32), pos])
    consts = jnp.concatenate([
        jnp.full((_L,), year_mean, dtype=jnp.float32),
        jnp.full((_L,), year_std, dtype=jnp.float32),
    ])
    out = k(isbn_idx, author_idx, year_of_publication, book_table,
            author_table, year_table, bpad, consts)
    return out.reshape(B, 3 * D + 1)
